# Initial kernel scaffold; baseline (speedup 1.0000x reference)
import jax
import jax.numpy as jnp
from jax.experimental import pallas as pl


def kernel(x, edge_index, statistical, batch, W1, b1, W2, b2, W3, b3, Ws, bs, Wc, bc):
    raise NotImplementedError("write your pallas kernel here")


# speedup vs baseline: 10.7438x; 10.7438x over previous
import functools

import jax
import jax.numpy as jnp
from jax import lax
from jax.experimental import pallas as pl
from jax.experimental.pallas import tpu as pltpu
from jax.experimental.pallas import tpu_sc as plsc

N = 10000
D = 128
H = 128
G = 64
S = 32
C = 2
E = 320000

NC = 2
NS = 16
NW = NC * NS

K = 128
EP = 331776
PT = EP // NW
NCHUNK = PT // K

NPAD = 10240
ZR = NPAD // NS

_mesh = plsc.VectorSubcoreMesh(
    core_axis_name="c", subcore_axis_name="s", num_cores=NC, num_subcores=NS)


@functools.partial(
    pl.kernel,
    out_type=jax.ShapeDtypeStruct((NC, NPAD, 16), jnp.float32),
    mesh=_mesh,
    scratch_types=[
        pltpu.VMEM((K,), jnp.int32),
        pltpu.VMEM((K, 16), jnp.float32),
        pltpu.VMEM_SHARED((NPAD, 16), jnp.float32),
    ],
)
def _sc_degree(dst_hbm, zeros16_hbm, ones16_hbm, out_hbm, idx_v, ones_v, acc_sh):
    c = lax.axis_index("c")
    s = lax.axis_index("s")
    wid = c * NS + s
    pltpu.sync_copy(zeros16_hbm, acc_sh.at[pl.ds(s * ZR, ZR)])
    pltpu.sync_copy(ones16_hbm, ones_v)
    plsc.subcore_barrier()

    def chunk(i, _):
        base = wid * PT + i * K
        pltpu.sync_copy(dst_hbm.at[pl.ds(base, K)], idx_v)
        pltpu.sync_copy(ones_v, acc_sh.at[idx_v], add=True)
        return _

    lax.fori_loop(0, NCHUNK, chunk, None)
    plsc.subcore_barrier()
    pltpu.sync_copy(acc_sh.at[pl.ds(s * ZR, ZR)], out_hbm.at[c, pl.ds(s * ZR, ZR)])


@functools.partial(
    pl.kernel,
    out_type=jax.ShapeDtypeStruct((NC, NPAD, H), jnp.float32),
    mesh=_mesh,
    scratch_types=[
        pltpu.VMEM((K,), jnp.int32),
        pltpu.VMEM((K,), jnp.int32),
        pltpu.VMEM((K, H), jnp.float32),
        pltpu.SemaphoreType.DMA,
        pltpu.VMEM_SHARED((NPAD, H), jnp.float32),
    ],
)
def _sc_scatter(hs_hbm, src_hbm, dst_hbm, zeros_hbm, out_hbm,
                src_v, dst_v, rows_v, sem, acc_sh):
    c = lax.axis_index("c")
    s = lax.axis_index("s")
    wid = c * NS + s
    pltpu.sync_copy(zeros_hbm, acc_sh.at[pl.ds(s * ZR, ZR)])
    plsc.subcore_barrier()

    def chunk(i, _):
        base = wid * PT + i * K
        pltpu.sync_copy(src_hbm.at[pl.ds(base, K)], src_v)
        pltpu.sync_copy(dst_hbm.at[pl.ds(base, K)], dst_v)
        pltpu.async_copy(hs_hbm.at[src_v], rows_v, sem).wait()
        pltpu.sync_copy(rows_v, acc_sh.at[dst_v], add=True)
        return _

    lax.fori_loop(0, NCHUNK, chunk, None)
    plsc.subcore_barrier()
    pltpu.sync_copy(acc_sh.at[pl.ds(s * ZR, ZR)], out_hbm.at[c, pl.ds(s * ZR, ZR)])


def _tc_pre_body(d0_ref, d1_ref, x_ref, w1_ref, stat_ref, ws_ref, bs_ref,
                 dinv_ref, hs1_ref, statout_ref):
    deg = d0_ref[:N, 0:1] + d1_ref[:N, 0:1]
    dinv = lax.rsqrt(deg)
    dinv_ref[...] = dinv
    h = jnp.dot(x_ref[...], w1_ref[...], preferred_element_type=jnp.float32)
    hs1_ref[...] = h * dinv
    st = jnp.dot(stat_ref[...], ws_ref[...], preferred_element_type=jnp.float32)
    statout_ref[...] = jnp.maximum(st + bs_ref[...], 0.0)


def _tc_mid_body(a0_ref, a1_ref, dinv_ref, b_ref, wn_ref, batch_ref,
                 hsn_ref, pool_ref):
    dinv = dinv_ref[...]
    xl = dinv * (a0_ref[:N, :] + a1_ref[:N, :]) + b_ref[...]
    xr = jnp.maximum(xl, 0.0)
    gid = lax.broadcasted_iota(jnp.int32, (N, G), 1)
    onehot = jnp.where(batch_ref[...] == gid, 1.0, 0.0)
    pool_ref[...] = lax.dot_general(
        onehot, xr, (((0,), (0,)), ((), ())),
        preferred_element_type=jnp.float32)
    hsn_ref[...] = jnp.dot(
        xl, wn_ref[...], preferred_element_type=jnp.float32) * dinv


def _tc_final_body(a0_ref, a1_ref, dinv_ref, b3_ref, batch_ref, p1_ref,
                   p2_ref, stat_ref, wc_ref, bc_ref, out_ref):
    dinv = dinv_ref[...]
    x3 = dinv * (a0_ref[:N, :] + a1_ref[:N, :]) + b3_ref[...]
    x3r = jnp.maximum(x3, 0.0)
    gid = lax.broadcasted_iota(jnp.int32, (N, G), 1)
    onehot = jnp.where(batch_ref[...] == gid, 1.0, 0.0)
    pool3 = lax.dot_general(onehot, x3r, (((0,), (0,)), ((), ())),
                            preferred_element_type=jnp.float32)
    cnt = lax.dot_general(onehot, jnp.ones((N, 1), jnp.float32),
                          (((0,), (0,)), ((), ())),
                          preferred_element_type=jnp.float32)
    rc = 1.0 / jnp.maximum(cnt, 1.0)
    comb = jnp.concatenate(
        [p1_ref[...] * rc, p2_ref[...] * rc, pool3 * rc, stat_ref[...]], axis=1)
    out_ref[...] = jnp.dot(
        comb, wc_ref[...], preferred_element_type=jnp.float32) + bc_ref[...]


_f32 = jnp.float32

_tc_pre = pl.pallas_call(
    _tc_pre_body,
    out_shape=[
        jax.ShapeDtypeStruct((N, 1), _f32),
        jax.ShapeDtypeStruct((N, H), _f32),
        jax.ShapeDtypeStruct((G, H), _f32),
    ],
)

_tc_mid = pl.pallas_call(
    _tc_mid_body,
    out_shape=[
        jax.ShapeDtypeStruct((N, H), _f32),
        jax.ShapeDtypeStruct((G, H), _f32),
    ],
)

_tc_final = pl.pallas_call(
    _tc_final_body,
    out_shape=jax.ShapeDtypeStruct((G, C), _f32),
)


def kernel(x, edge_index, statistical, batch, W1, b1, W2, b2, W3, b3,
           Ws, bs, Wc, bc):
    ei = edge_index.astype(jnp.int32)
    si = jnp.arange(N, dtype=jnp.int32)
    npadjunk = EP - E - N
    src = jnp.concatenate([ei[0], si, jnp.zeros((npadjunk,), jnp.int32)])
    dst = jnp.concatenate([ei[1], si, jnp.full((npadjunk,), N, jnp.int32)])

    zeros16 = jnp.zeros((ZR, 16), _f32)
    ones16 = jnp.ones((K, 16), _f32)
    zrows = jnp.zeros((ZR, H), _f32)
    batch2d = batch.astype(jnp.int32).reshape(N, 1)

    degp = _sc_degree(dst, zeros16, ones16)
    dinv, hs1, stat = _tc_pre(degp[0], degp[1], x, W1, statistical, Ws,
                              bs.reshape(1, H))

    acc1 = _sc_scatter(hs1, src, dst, zrows)
    hs2, pool1 = _tc_mid(acc1[0], acc1[1], dinv, b1.reshape(1, H), W2, batch2d)

    acc2 = _sc_scatter(hs2, src, dst, zrows)
    hs3, pool2 = _tc_mid(acc2[0], acc2[1], dinv, b2.reshape(1, H), W3, batch2d)

    acc3 = _sc_scatter(hs3, src, dst, zrows)
    out = _tc_final(acc3[0], acc3[1], dinv, b3.reshape(1, H), batch2d,
                    pool1, pool2, stat, Wc, bc.reshape(1, C))
    return out
